# baseline (speedup 1.0000x reference)
import jax
import jax.numpy as jnp
from jax import lax
from jax.experimental import pallas as pl
from jax.experimental.pallas import tpu as pltpu
from jax.experimental.pallas import tpu_sc as plsc

NU = 25000
D = 64
HALF = 25088
NC = 2
NT = 16
CHK = 96
BCH = 12
NBLK = 22
NCH = NBLK * BCH
EPT = NCH * CHK
ROWS_PT = HALF // NT
RCHK = 128
NFULL = ROWS_PT // RCHK
REM = ROWS_PT - NFULL * RCHK
LFULL = ROWS_PT // CHK
LREM = ROWS_PT - LFULL * CHK
FCHK = 128
F32 = jnp.float32
I32 = jnp.int32

_MESH = dict(core_axis_name="c", subcore_axis_name="s",
             num_cores=NC, num_subcores=NT)


def _mesh():
    return plsc.VectorSubcoreMesh(**_MESH)


def _params():
    return pltpu.CompilerParams(use_tc_tiling_on_sc=False,
                                needs_layout_passes=False)


def _rsqrt16(x):
    m = jnp.maximum(x, 1.0)
    s = jnp.full((16,), 1.0, F32)
    for _ in range(10):
        big = m > 4.0
        m = jnp.where(big, m * 0.25, m)
        s = jnp.where(big, s * 0.5, s)
    y = 1.1667 - 0.1667 * m
    for _ in range(4):
        y = y * (1.5 - 0.5 * m * y * y)
    return jnp.where(x < 0.5, 0.0, y * s)


def _zero_rows(buf, n):
    w = buf.shape[1]

    def body(i, _):
        for k in range(w // 16):
            buf[i, 16 * k:16 * (k + 1)] = jnp.zeros((16,), F32)
        return 0

    lax.fori_loop(0, n, body, 0)


def _blocked_idx_sweep(c, s, idx_hbms, idx_bufs, sem_i, do_block):
    def load(b, par):
        for h, v in zip(idx_hbms, idx_bufs):
            pltpu.async_copy(h.at[c, s, b], v.at[par], sem_i)

    def wait(b, par):
        for h, v in zip(idx_hbms, idx_bufs):
            pltpu.make_async_copy(h.at[c, s, b], v.at[par], sem_i).wait()

    load(0, 0)
    wait(0, 0)

    def bpair(t, _):
        b = 2 * t
        load(b + 1, 1)
        do_block(0)
        wait(b + 1, 1)

        @pl.when(b + 2 < NBLK)
        def _():
            load(b + 2, 0)

        do_block(1)

        @pl.when(b + 2 < NBLK)
        def _():
            wait(b + 2, 0)

        return 0

    lax.fori_loop(0, NBLK // 2, bpair, 0)


def _init_body(dstg, xp, dinvb, z0, dg_sp, idxd_v, ones_v, dv_a, dv_b,
               xb_a, xb_b, sem_i, sem_ra, sem_rb, sem_wa, sem_wb,
               sem_ra2, sem_rb2, sem_wa2, sem_wb2, sem_s):
    c = lax.axis_index("c")
    s = lax.axis_index("s")
    r0 = s * ROWS_PT

    _zero_rows(dv_a, RCHK)

    def fill_ones(i, _):
        ones_v[i] = jnp.full((16,), 1.0, F32)
        return 0

    lax.fori_loop(0, CHK, fill_ones, 0)

    def zchunk(k, _):
        pltpu.sync_copy(dv_a, dg_sp.at[pl.ds(r0 + k * RCHK, RCHK)])
        return 0

    lax.fori_loop(0, NFULL, zchunk, 0)
    pltpu.sync_copy(dv_a.at[pl.ds(0, REM)],
                    dg_sp.at[pl.ds(r0 + NFULL * RCHK, REM)])
    plsc.subcore_barrier()

    def do_block(par):
        def chunk(j, _):
            pltpu.async_copy(ones_v, dg_sp.at[idxd_v.at[par, j]], sem_s,
                             add=True)
            return 0

        lax.fori_loop(0, BCH, chunk, 0)

        def drain(j, _):
            pltpu.make_async_copy(ones_v, dg_sp.at[idxd_v.at[par, j]],
                                  sem_s).wait()
            return 0

        lax.fori_loop(0, BCH, drain, 0)

    _blocked_idx_sweep(c, s, [dstg], [idxd_v], sem_i, do_block)
    plsc.subcore_barrier()

    chunks = [(r0 + k * RCHK, RCHK) for k in range(NFULL)]
    chunks.append((r0 + NFULL * RCHK, REM))
    bufs = [(dv_a, xb_a, sem_ra, sem_wa, sem_ra2, sem_wa2),
            (dv_b, xb_b, sem_rb, sem_wb, sem_rb2, sem_wb2)]

    def read(i):
        base, n = chunks[i]
        dv, xb, sr, _, sr2, _ = bufs[i % 2]
        pltpu.async_copy(dg_sp.at[pl.ds(base, n)], dv.at[pl.ds(0, n)], sr)
        pltpu.async_copy(xp.at[pl.ds(c * HALF + base, n)],
                         xb.at[pl.ds(0, n)], sr2)

    def wait_read(i):
        base, n = chunks[i]
        dv, xb, sr, _, sr2, _ = bufs[i % 2]
        pltpu.make_async_copy(
            dg_sp.at[pl.ds(base, n)], dv.at[pl.ds(0, n)], sr).wait()
        pltpu.make_async_copy(
            xp.at[pl.ds(c * HALF + base, n)], xb.at[pl.ds(0, n)],
            sr2).wait()

    def write(i):
        base, n = chunks[i]
        dv, xb, _, sw, _, sw2 = bufs[i % 2]
        pltpu.async_copy(dv.at[pl.ds(0, n)], dinvb.at[c, pl.ds(base, n)],
                         sw)
        pltpu.async_copy(xb.at[pl.ds(0, n)],
                         z0.at[pl.ds(c * HALF + base, n)], sw2)

    def wait_write(i):
        base, n = chunks[i]
        dv, xb, _, sw, _, sw2 = bufs[i % 2]
        pltpu.make_async_copy(
            dv.at[pl.ds(0, n)], dinvb.at[c, pl.ds(base, n)], sw).wait()
        pltpu.make_async_copy(
            xb.at[pl.ds(0, n)], z0.at[pl.ds(c * HALF + base, n)],
            sw2).wait()

    def compute(i):
        _, n = chunks[i]
        dv, xb = bufs[i % 2][0], bufs[i % 2][1]

        def row(r, _):
            d = _rsqrt16(dv[r])
            dv[r] = d
            for k in range(4):
                sl = slice(16 * k, 16 * (k + 1))
                xb[r, sl] = xb[r, sl] * d
            return 0

        lax.fori_loop(0, n, row, 0)

    read(0)
    for i in range(len(chunks)):
        if i + 1 < len(chunks):
            if i >= 1:
                wait_write(i - 1)
            read(i + 1)
        wait_read(i)
        compute(i)
        write(i)
    wait_write(len(chunks) - 2)
    wait_write(len(chunks) - 1)


def _layer_body(srcg, dstg, z_in, dinvb, z_out, acc_sp,
                idxs_v, idxd_v, buf_a, buf_b, buf_c, dv_a, dv_b,
                sem_g0, sem_g1, sem_g2, sem_s0, sem_s1, sem_s2, sem_i,
                sem_ra, sem_rb, sem_wa, sem_wb, sem_ra2, sem_rb2):
    c = lax.axis_index("c")
    s = lax.axis_index("s")
    r0 = s * ROWS_PT
    gbuf = [buf_a, buf_b, buf_c]
    sg = [sem_g0, sem_g1, sem_g2]
    ss = [sem_s0, sem_s1, sem_s2]

    _zero_rows(buf_a, CHK)

    def zchunk(k, _):
        pltpu.sync_copy(buf_a, acc_sp.at[pl.ds(r0 + k * CHK, CHK)])
        return 0

    lax.fori_loop(0, LFULL, zchunk, 0)
    pltpu.sync_copy(buf_a.at[pl.ds(0, LREM)],
                    acc_sp.at[pl.ds(r0 + LFULL * CHK, LREM)])
    plsc.subcore_barrier()

    def do_block(par):
        def gath(ch, b):
            pltpu.async_copy(z_in.at[idxs_v.at[par, ch]], gbuf[b], sg[b])

        def gwait(ch, b):
            pltpu.make_async_copy(
                z_in.at[idxs_v.at[par, ch]], gbuf[b], sg[b]).wait()

        def scat(ch, b):
            pltpu.async_copy(gbuf[b], acc_sp.at[idxd_v.at[par, ch]],
                             ss[b], add=True)

        def swait(ch, b):
            pltpu.make_async_copy(
                gbuf[b], acc_sp.at[idxd_v.at[par, ch]], ss[b]).wait()

        gath(0, 0)
        gath(1, 1)
        for ch in range(BCH):
            b = ch % 3
            gwait(ch, b)
            scat(ch, b)
            if ch == 0:
                gath(2, 2)
            elif ch + 2 < BCH:
                b2 = (ch + 2) % 3
                swait(ch - 1, b2)
                gath(ch + 2, b2)
            elif ch + 2 == BCH:
                swait(ch - 1, (ch + 2) % 3)
        swait(BCH - 2, (BCH - 2) % 3)
        swait(BCH - 1, (BCH - 1) % 3)

    _blocked_idx_sweep(c, s, [srcg, dstg], [idxs_v, idxd_v], sem_i, do_block)
    plsc.subcore_barrier()

    chunks = [(r0 + k * CHK, CHK) for k in range(LFULL)]
    chunks.append((r0 + LFULL * CHK, LREM))
    bufs = [(dv_a, buf_a, sem_ra, sem_wa, sem_ra2),
            (dv_b, buf_b, sem_rb, sem_wb, sem_rb2)]

    def read(i):
        base, n = chunks[i]
        dv, xb, sr, _, sr2 = bufs[i % 2]
        pltpu.async_copy(acc_sp.at[pl.ds(base, n)], xb.at[pl.ds(0, n)], sr)
        pltpu.async_copy(dinvb.at[c, pl.ds(base, n)], dv.at[pl.ds(0, n)],
                         sr2)

    def wait_read(i):
        base, n = chunks[i]
        dv, xb, sr, _, sr2 = bufs[i % 2]
        pltpu.make_async_copy(
            acc_sp.at[pl.ds(base, n)], xb.at[pl.ds(0, n)], sr).wait()
        pltpu.make_async_copy(
            dinvb.at[c, pl.ds(base, n)], dv.at[pl.ds(0, n)], sr2).wait()

    def write(i):
        base, n = chunks[i]
        xb, sw = bufs[i % 2][1], bufs[i % 2][3]
        pltpu.async_copy(xb.at[pl.ds(0, n)],
                         z_out.at[pl.ds(c * HALF + base, n)], sw)

    def wait_write(i):
        base, n = chunks[i]
        xb, sw = bufs[i % 2][1], bufs[i % 2][3]
        pltpu.make_async_copy(
            xb.at[pl.ds(0, n)], z_out.at[pl.ds(c * HALF + base, n)],
            sw).wait()

    def compute(i):
        _, n = chunks[i]
        dv, xb = bufs[i % 2][0], bufs[i % 2][1]

        def row(r, _):
            d = dv[r]
            d2 = d * d
            for k in range(4):
                sl = slice(16 * k, 16 * (k + 1))
                xb[r, sl] = xb[r, sl] * d2
            return 0

        lax.fori_loop(0, n, row, 0)

    read(0)
    for i in range(len(chunks)):
        if i + 1 < len(chunks):
            if i >= 1:
                wait_write(i - 1)
            read(i + 1)
        wait_read(i)
        compute(i)
        write(i)
    wait_write(len(chunks) - 2)
    wait_write(len(chunks) - 1)


def _final_body(xp, z1, z2, z3, dvf, ur, ir, gamma, idxu_v, idxi_v,
                x0u_v, z1u_v, z2u_v, z3u_v, dvu_v,
                x0i_v, z1i_v, z2i_v, z3i_v, dvi_v, g_v, sem_g):
    c = lax.axis_index("c")
    s = lax.axis_index("s")
    pltpu.sync_copy(ur.at[c, s], idxu_v)
    pltpu.sync_copy(ir.at[c, s], idxi_v)
    lane = lax.iota(I32, 16)

    tabs = [xp, z1, z2, z3, dvf]
    ubufs = [x0u_v, z1u_v, z2u_v, z3u_v, dvu_v]
    ibufs = [x0i_v, z1i_v, z2i_v, z3i_v, dvi_v]

    def gather(ch):
        for t, b in zip(tabs, ubufs):
            pltpu.async_copy(t.at[idxu_v.at[ch]], b, sem_g)
        for t, b in zip(tabs, ibufs):
            pltpu.async_copy(t.at[idxi_v.at[ch]], b, sem_g)

    def drain(ch):
        for t, b in zip(tabs, ubufs):
            pltpu.make_async_copy(t.at[idxu_v.at[ch]], b, sem_g).wait()
        for t, b in zip(tabs, ibufs):
            pltpu.make_async_copy(t.at[idxi_v.at[ch]], b, sem_g).wait()

    gather(0)

    def ch_body(ch, _):
        drain(ch)

        def group(g, _):
            def pair(j, res):
                p = g * 16 + j
                du = dvu_v[p, 0:16]
                di = dvi_v[p, 0:16]
                ru = jnp.where(du > 0.0, 1.0 / jnp.maximum(du, 1e-30), 0.0)
                ri = jnp.where(di > 0.0, 1.0 / jnp.maximum(di, 1e-30), 0.0)
                acc = jnp.zeros((16,), F32)
                for k in range(4):
                    sl = slice(16 * k, 16 * (k + 1))
                    zu = z1u_v[p, sl] + z2u_v[p, sl] + z3u_v[p, sl]
                    zi = z1i_v[p, sl] + z2i_v[p, sl] + z3i_v[p, sl]
                    au = x0u_v[p, sl] + zu * ru
                    ai = x0i_v[p, sl] + zi * ri
                    acc = acc + au * ai
                return jnp.where(lane == j, jnp.sum(acc) * 0.0625, res)

            g_v[ch * 8 + g] = lax.fori_loop(0, 16, pair, jnp.zeros((16,), F32))
            return 0

        lax.fori_loop(0, 8, group, 0)

        @pl.when(ch + 1 < 4)
        def _():
            gather_next(ch)

        return 0

    def gather_next(ch):
        for t, b in zip(tabs, ubufs):
            pltpu.async_copy(t.at[idxu_v.at[ch + 1]], b, sem_g)
        for t, b in zip(tabs, ibufs):
            pltpu.async_copy(t.at[idxi_v.at[ch + 1]], b, sem_g)

    lax.fori_loop(0, 4, ch_body, 0)
    w = c * NT + s
    pltpu.sync_copy(g_v, gamma.at[pl.ds(w * 32, 32)])


def _make_init():
    return pl.kernel(
        _init_body,
        out_type=[jax.ShapeDtypeStruct((NC, HALF, 16), F32),
                  jax.ShapeDtypeStruct((NC * HALF, D), F32)],
        mesh=_mesh(),
        compiler_params=_params(),
        scratch_types=[
            pltpu.VMEM_SHARED((HALF, 16), F32),
            pltpu.VMEM((2, BCH, CHK), I32),
            pltpu.VMEM((CHK, 16), F32),
            pltpu.VMEM((RCHK, 16), F32),
            pltpu.VMEM((RCHK, 16), F32),
            pltpu.VMEM((RCHK, D), F32),
            pltpu.VMEM((RCHK, D), F32),
        ] + [pltpu.SemaphoreType.DMA] * 10,
    )


def _make_layer():
    return pl.kernel(
        _layer_body,
        out_type=jax.ShapeDtypeStruct((NC * HALF, D), F32),
        mesh=_mesh(),
        compiler_params=_params(),
        scratch_types=[
            pltpu.VMEM_SHARED((HALF, D), F32),
            pltpu.VMEM((2, BCH, CHK), I32),
            pltpu.VMEM((2, BCH, CHK), I32),
            pltpu.VMEM((CHK, D), F32),
            pltpu.VMEM((CHK, D), F32),
            pltpu.VMEM((CHK, D), F32),
            pltpu.VMEM((CHK, 16), F32),
            pltpu.VMEM((CHK, 16), F32),
        ] + [pltpu.SemaphoreType.DMA] * 13,
    )


def _make_final():
    return pl.kernel(
        _final_body,
        out_type=jax.ShapeDtypeStruct((NC * NT * 32, 16), F32),
        mesh=_mesh(),
        compiler_params=_params(),
        scratch_types=[
            pltpu.VMEM((4, FCHK), I32),
            pltpu.VMEM((4, FCHK), I32),
            pltpu.VMEM((FCHK, D), F32),
            pltpu.VMEM((FCHK, D), F32),
            pltpu.VMEM((FCHK, D), F32),
            pltpu.VMEM((FCHK, D), F32),
            pltpu.VMEM((FCHK, 16), F32),
            pltpu.VMEM((FCHK, D), F32),
            pltpu.VMEM((FCHK, D), F32),
            pltpu.VMEM((FCHK, D), F32),
            pltpu.VMEM((FCHK, D), F32),
            pltpu.VMEM((FCHK, 16), F32),
            pltpu.VMEM((32, 16), F32),
            pltpu.SemaphoreType.DMA,
        ],
    )


def kernel(users, items, edge_index, user_emb, item_emb):
    src = edge_index[0].astype(I32)
    dst = edge_index[1].astype(I32)
    eh = src.shape[0] // 2

    src_adj = src + (HALF - NU) * (src >= NU).astype(I32)
    pad_n = NT * EPT - eh

    def prep(sa, dl):
        sa = jnp.concatenate([sa, jnp.zeros((pad_n,), I32)])
        dl = jnp.concatenate([dl, jnp.full((pad_n,), NU, I32)])
        return (sa.reshape(NT, NBLK, BCH, CHK),
                dl.reshape(NT, NBLK, BCH, CHK))

    s0, d0 = prep(src_adj[eh:], dst[eh:])
    s1, d1 = prep(src_adj[:eh], dst[:eh] - NU)
    srcg = jnp.stack([s0, s1])
    dstg = jnp.stack([d0, d1])

    zpad = jnp.zeros((HALF - NU, D), F32)
    xp = jnp.concatenate([user_emb.astype(F32), zpad,
                          item_emb.astype(F32), zpad], axis=0)
    ur = users.astype(I32).reshape(NC, NT, 4, FCHK)
    ir = (items.astype(I32) + HALF).reshape(NC, NT, 4, FCHK)

    dinvb, z0 = _make_init()(dstg, xp)
    layer = _make_layer()
    z1 = layer(srcg, dstg, z0, dinvb)
    z2 = layer(srcg, dstg, z1, dinvb)
    z3 = layer(srcg, dstg, z2, dinvb)
    dvf = dinvb.reshape(NC * HALF, 16)
    gamma = _make_final()(xp, z1, z2, z3, dvf, ur, ir)
    return gamma.reshape(-1)


# speedup vs baseline: 1.4941x; 1.4941x over previous
import jax
import jax.numpy as jnp
from jax import lax
from jax.experimental import pallas as pl
from jax.experimental.pallas import tpu as pltpu
from jax.experimental.pallas import tpu_sc as plsc

NU = 25000
D = 64
HALF = 25088
NC = 2
NT = 16
CHK = 128
BCH = 14
NBLK = 14
NCH = NBLK * BCH
EPT = NCH * CHK
ROWS_PT = HALF // NT
NFULL = ROWS_PT // CHK
REM = ROWS_PT - NFULL * CHK
F32 = jnp.float32
I32 = jnp.int32

_MESH = dict(core_axis_name="c", subcore_axis_name="s",
             num_cores=NC, num_subcores=NT)


def _mesh():
    return plsc.VectorSubcoreMesh(**_MESH)


def _params():
    return pltpu.CompilerParams(use_tc_tiling_on_sc=False,
                                needs_layout_passes=False)


def _rsqrt16(x):
    m = jnp.maximum(x, 1.0)
    s = jnp.full((16,), 1.0, F32)
    for _ in range(10):
        big = m > 4.0
        m = jnp.where(big, m * 0.25, m)
        s = jnp.where(big, s * 0.5, s)
    y = 1.1667 - 0.1667 * m
    for _ in range(4):
        y = y * (1.5 - 0.5 * m * y * y)
    return jnp.where(x < 0.5, 0.0, y * s)


def _zero_rows(buf, n):
    w = buf.shape[1]

    def body(i, _):
        for k in range(w // 16):
            buf[i, 16 * k:16 * (k + 1)] = jnp.zeros((16,), F32)
        return 0

    lax.fori_loop(0, n, body, 0)


def _blocked_idx_sweep(c, s, idx_hbms, idx_bufs, sem_i, do_block):
    def load(b, par):
        for h, v in zip(idx_hbms, idx_bufs):
            pltpu.async_copy(h.at[c, s, b], v.at[par], sem_i)

    def wait(b, par):
        for h, v in zip(idx_hbms, idx_bufs):
            pltpu.make_async_copy(h.at[c, s, b], v.at[par], sem_i).wait()

    load(0, 0)
    wait(0, 0)

    def bpair(t, _):
        b = 2 * t
        load(b + 1, 1)
        do_block(0)
        wait(b + 1, 1)

        @pl.when(b + 2 < NBLK)
        def _():
            load(b + 2, 0)

        do_block(1)

        @pl.when(b + 2 < NBLK)
        def _():
            wait(b + 2, 0)

        return 0

    lax.fori_loop(0, NBLK // 2, bpair, 0)


def _init_body(dstg, xp, dinvb, z0, dg_sp, idxd_v, ones_v, dv_a, dv_b,
               xb_a, xb_b, sem_i, sem_ra, sem_rb, sem_wa, sem_wb,
               sem_ra2, sem_rb2, sem_wa2, sem_wb2, sem_s):
    c = lax.axis_index("c")
    s = lax.axis_index("s")
    r0 = s * ROWS_PT

    _zero_rows(dv_a, CHK)

    def fill_ones(i, _):
        ones_v[i] = jnp.full((16,), 1.0, F32)
        return 0

    lax.fori_loop(0, CHK, fill_ones, 0)

    def zchunk(k, _):
        pltpu.sync_copy(dv_a, dg_sp.at[pl.ds(r0 + k * CHK, CHK)])
        return 0

    lax.fori_loop(0, NFULL, zchunk, 0)
    pltpu.sync_copy(dv_a.at[pl.ds(0, REM)],
                    dg_sp.at[pl.ds(r0 + NFULL * CHK, REM)])
    plsc.subcore_barrier()

    def do_block(par):
        def chunk(j, _):
            pltpu.async_copy(ones_v, dg_sp.at[idxd_v.at[par, j]], sem_s,
                             add=True)
            return 0

        lax.fori_loop(0, BCH, chunk, 0)

        def drain(j, _):
            pltpu.make_async_copy(ones_v, dg_sp.at[idxd_v.at[par, j]],
                                  sem_s).wait()
            return 0

        lax.fori_loop(0, BCH, drain, 0)

    _blocked_idx_sweep(c, s, [dstg], [idxd_v], sem_i, do_block)
    plsc.subcore_barrier()

    chunks = [(r0 + k * CHK, CHK) for k in range(NFULL)]
    chunks.append((r0 + NFULL * CHK, REM))
    bufs = [(dv_a, xb_a, sem_ra, sem_wa, sem_ra2, sem_wa2),
            (dv_b, xb_b, sem_rb, sem_wb, sem_rb2, sem_wb2)]

    def read(i):
        base, n = chunks[i]
        dv, xb, sr, _, sr2, _ = bufs[i % 2]
        pltpu.async_copy(dg_sp.at[pl.ds(base, n)], dv.at[pl.ds(0, n)], sr)
        pltpu.async_copy(xp.at[pl.ds(c * HALF + base, n)],
                         xb.at[pl.ds(0, n)], sr2)

    def wait_read(i):
        base, n = chunks[i]
        dv, xb, sr, _, sr2, _ = bufs[i % 2]
        pltpu.make_async_copy(
            dg_sp.at[pl.ds(base, n)], dv.at[pl.ds(0, n)], sr).wait()
        pltpu.make_async_copy(
            xp.at[pl.ds(c * HALF + base, n)], xb.at[pl.ds(0, n)],
            sr2).wait()

    def write(i):
        base, n = chunks[i]
        dv, xb, _, sw, _, sw2 = bufs[i % 2]
        pltpu.async_copy(dv.at[pl.ds(0, n)], dinvb.at[c, pl.ds(base, n)],
                         sw)
        pltpu.async_copy(xb.at[pl.ds(0, n)],
                         z0.at[pl.ds(c * HALF + base, n)], sw2)

    def wait_write(i):
        base, n = chunks[i]
        dv, xb, _, sw, _, sw2 = bufs[i % 2]
        pltpu.make_async_copy(
            dv.at[pl.ds(0, n)], dinvb.at[c, pl.ds(base, n)], sw).wait()
        pltpu.make_async_copy(
            xb.at[pl.ds(0, n)], z0.at[pl.ds(c * HALF + base, n)],
            sw2).wait()

    def compute(i):
        _, n = chunks[i]
        dv, xb = bufs[i % 2][0], bufs[i % 2][1]

        def row(r, _):
            d = _rsqrt16(dv[r])
            dv[r] = d
            for k in range(4):
                sl = slice(16 * k, 16 * (k + 1))
                xb[r, sl] = xb[r, sl] * d
            return 0

        lax.fori_loop(0, n, row, 0)

    read(0)
    for i in range(len(chunks)):
        if i + 1 < len(chunks):
            if i >= 1:
                wait_write(i - 1)
            read(i + 1)
        wait_read(i)
        compute(i)
        write(i)
    wait_write(len(chunks) - 2)
    wait_write(len(chunks) - 1)


def _layer_body(srcg, dstg, z_in, dinvb, z_out, acc_sp,
                idxs_v, idxd_v, buf_a, buf_b, dv_a, dv_b,
                sem_a, sem_b, sem_i, sem_ra, sem_rb, sem_wa, sem_wb,
                sem_ra2, sem_rb2, sem_sa, sem_sb):
    c = lax.axis_index("c")
    s = lax.axis_index("s")
    r0 = s * ROWS_PT
    layer_i = 0
    _zero_rows(buf_a, CHK)

    def zchunk(k, _):
        pltpu.sync_copy(buf_a, acc_sp.at[pl.ds(r0 + k * CHK, CHK)])
        return 0

    lax.fori_loop(0, NFULL, zchunk, 0)
    pltpu.sync_copy(buf_a.at[pl.ds(0, REM)],
                    acc_sp.at[pl.ds(r0 + NFULL * CHK, REM)])
    plsc.subcore_barrier()
    del layer_i

    def do_block(par):
        pltpu.async_copy(z_in.at[idxs_v.at[par, 0]], buf_a, sem_a)

        def pair(j, _):
            ch = 2 * j

            @pl.when(j > 0)
            def _():
                pltpu.make_async_copy(
                    buf_b, acc_sp.at[idxd_v.at[par, ch - 1]], sem_sb).wait()

            pltpu.async_copy(z_in.at[idxs_v.at[par, ch + 1]], buf_b, sem_b)
            pltpu.make_async_copy(
                z_in.at[idxs_v.at[par, ch]], buf_a, sem_a).wait()
            pltpu.async_copy(buf_a, acc_sp.at[idxd_v.at[par, ch]], sem_sa,
                             add=True)

            @pl.when(ch + 2 < BCH)
            def _():
                pltpu.make_async_copy(
                    buf_a, acc_sp.at[idxd_v.at[par, ch]], sem_sa).wait()
                pltpu.async_copy(
                    z_in.at[idxs_v.at[par, ch + 2]], buf_a, sem_a)

            pltpu.make_async_copy(
                z_in.at[idxs_v.at[par, ch + 1]], buf_b, sem_b).wait()
            pltpu.async_copy(buf_b, acc_sp.at[idxd_v.at[par, ch + 1]],
                             sem_sb, add=True)
            return 0

        lax.fori_loop(0, BCH // 2, pair, 0)
        pltpu.make_async_copy(
            buf_a, acc_sp.at[idxd_v.at[par, BCH - 2]], sem_sa).wait()
        pltpu.make_async_copy(
            buf_b, acc_sp.at[idxd_v.at[par, BCH - 1]], sem_sb).wait()

    _blocked_idx_sweep(c, s, [srcg, dstg], [idxs_v, idxd_v], sem_i, do_block)
    plsc.subcore_barrier()

    chunks = [(r0 + k * CHK, CHK) for k in range(NFULL)]
    chunks.append((r0 + NFULL * CHK, REM))
    bufs = [(dv_a, buf_a, sem_ra, sem_wa, sem_ra2),
            (dv_b, buf_b, sem_rb, sem_wb, sem_rb2)]

    def read(i):
        base, n = chunks[i]
        dv, xb, sr, _, sr2 = bufs[i % 2]
        pltpu.async_copy(acc_sp.at[pl.ds(base, n)], xb.at[pl.ds(0, n)], sr)
        pltpu.async_copy(dinvb.at[c, pl.ds(base, n)], dv.at[pl.ds(0, n)],
                         sr2)

    def wait_read(i):
        base, n = chunks[i]
        dv, xb, sr, _, sr2 = bufs[i % 2]
        pltpu.make_async_copy(
            acc_sp.at[pl.ds(base, n)], xb.at[pl.ds(0, n)], sr).wait()
        pltpu.make_async_copy(
            dinvb.at[c, pl.ds(base, n)], dv.at[pl.ds(0, n)], sr2).wait()

    def write(i):
        base, n = chunks[i]
        xb, sw = bufs[i % 2][1], bufs[i % 2][3]
        pltpu.async_copy(xb.at[pl.ds(0, n)],
                         z_out.at[pl.ds(c * HALF + base, n)], sw)

    def wait_write(i):
        base, n = chunks[i]
        xb, sw = bufs[i % 2][1], bufs[i % 2][3]
        pltpu.make_async_copy(
            xb.at[pl.ds(0, n)], z_out.at[pl.ds(c * HALF + base, n)],
            sw).wait()

    def compute(i):
        _, n = chunks[i]
        dv, xb = bufs[i % 2][0], bufs[i % 2][1]

        def row(r, _):
            d = dv[r]
            d2 = d * d
            for k in range(4):
                sl = slice(16 * k, 16 * (k + 1))
                xb[r, sl] = xb[r, sl] * d2
            return 0

        lax.fori_loop(0, n, row, 0)

    read(0)
    for i in range(len(chunks)):
        if i + 1 < len(chunks):
            if i >= 1:
                wait_write(i - 1)
            read(i + 1)
        wait_read(i)
        compute(i)
        write(i)
    wait_write(len(chunks) - 2)
    wait_write(len(chunks) - 1)


def _final_body(xp, z1, z2, z3, dvf, ur, ir, gamma, idxu_v, idxi_v,
                x0u_v, z1u_v, z2u_v, z3u_v, dvu_v,
                x0i_v, z1i_v, z2i_v, z3i_v, dvi_v, g_v, sem_g):
    c = lax.axis_index("c")
    s = lax.axis_index("s")
    pltpu.sync_copy(ur.at[c, s], idxu_v)
    pltpu.sync_copy(ir.at[c, s], idxi_v)
    lane = lax.iota(I32, 16)

    tabs = [xp, z1, z2, z3, dvf]
    ubufs = [x0u_v, z1u_v, z2u_v, z3u_v, dvu_v]
    ibufs = [x0i_v, z1i_v, z2i_v, z3i_v, dvi_v]

    def gather(ch):
        for t, b in zip(tabs, ubufs):
            pltpu.async_copy(t.at[idxu_v.at[ch]], b, sem_g)
        for t, b in zip(tabs, ibufs):
            pltpu.async_copy(t.at[idxi_v.at[ch]], b, sem_g)

    def drain(ch):
        for t, b in zip(tabs, ubufs):
            pltpu.make_async_copy(t.at[idxu_v.at[ch]], b, sem_g).wait()
        for t, b in zip(tabs, ibufs):
            pltpu.make_async_copy(t.at[idxi_v.at[ch]], b, sem_g).wait()

    gather(0)

    def ch_body(ch, _):
        drain(ch)

        def group(g, _):
            def pair(j, res):
                p = g * 16 + j
                du = dvu_v[p, 0:16]
                di = dvi_v[p, 0:16]
                ru = jnp.where(du > 0.0, 1.0 / jnp.maximum(du, 1e-30), 0.0)
                ri = jnp.where(di > 0.0, 1.0 / jnp.maximum(di, 1e-30), 0.0)
                acc = jnp.zeros((16,), F32)
                for k in range(4):
                    sl = slice(16 * k, 16 * (k + 1))
                    zu = z1u_v[p, sl] + z2u_v[p, sl] + z3u_v[p, sl]
                    zi = z1i_v[p, sl] + z2i_v[p, sl] + z3i_v[p, sl]
                    au = x0u_v[p, sl] + zu * ru
                    ai = x0i_v[p, sl] + zi * ri
                    acc = acc + au * ai
                return jnp.where(lane == j, jnp.sum(acc) * 0.0625, res)

            g_v[ch * 8 + g] = lax.fori_loop(0, 16, pair, jnp.zeros((16,), F32))
            return 0

        lax.fori_loop(0, 8, group, 0)

        @pl.when(ch + 1 < 4)
        def _():
            gather_next(ch)

        return 0

    def gather_next(ch):
        for t, b in zip(tabs, ubufs):
            pltpu.async_copy(t.at[idxu_v.at[ch + 1]], b, sem_g)
        for t, b in zip(tabs, ibufs):
            pltpu.async_copy(t.at[idxi_v.at[ch + 1]], b, sem_g)

    lax.fori_loop(0, 4, ch_body, 0)
    w = c * NT + s
    pltpu.sync_copy(g_v, gamma.at[pl.ds(w * 32, 32)])


def _make_init():
    return pl.kernel(
        _init_body,
        out_type=[jax.ShapeDtypeStruct((NC, HALF, 16), F32),
                  jax.ShapeDtypeStruct((NC * HALF, D), F32)],
        mesh=_mesh(),
        compiler_params=_params(),
        scratch_types=[
            pltpu.VMEM_SHARED((HALF, 16), F32),
            pltpu.VMEM((2, BCH, CHK), I32),
            pltpu.VMEM((CHK, 16), F32),
            pltpu.VMEM((CHK, 16), F32),
            pltpu.VMEM((CHK, 16), F32),
            pltpu.VMEM((CHK, D), F32),
            pltpu.VMEM((CHK, D), F32),
        ] + [pltpu.SemaphoreType.DMA] * 10,
    )


def _make_layer():
    return pl.kernel(
        _layer_body,
        out_type=jax.ShapeDtypeStruct((NC * HALF, D), F32),
        mesh=_mesh(),
        compiler_params=_params(),
        scratch_types=[
            pltpu.VMEM_SHARED((HALF, D), F32),
            pltpu.VMEM((2, BCH, CHK), I32),
            pltpu.VMEM((2, BCH, CHK), I32),
            pltpu.VMEM((CHK, D), F32),
            pltpu.VMEM((CHK, D), F32),
            pltpu.VMEM((CHK, 16), F32),
            pltpu.VMEM((CHK, 16), F32),
        ] + [pltpu.SemaphoreType.DMA] * 11,
    )


def _make_final():
    return pl.kernel(
        _final_body,
        out_type=jax.ShapeDtypeStruct((NC * NT * 32, 16), F32),
        mesh=_mesh(),
        compiler_params=_params(),
        scratch_types=[
            pltpu.VMEM((4, CHK), I32),
            pltpu.VMEM((4, CHK), I32),
            pltpu.VMEM((CHK, D), F32),
            pltpu.VMEM((CHK, D), F32),
            pltpu.VMEM((CHK, D), F32),
            pltpu.VMEM((CHK, D), F32),
            pltpu.VMEM((CHK, 16), F32),
            pltpu.VMEM((CHK, D), F32),
            pltpu.VMEM((CHK, D), F32),
            pltpu.VMEM((CHK, D), F32),
            pltpu.VMEM((CHK, D), F32),
            pltpu.VMEM((CHK, 16), F32),
            pltpu.VMEM((32, 16), F32),
            pltpu.SemaphoreType.DMA,
        ],
    )


def kernel(users, items, edge_index, user_emb, item_emb):
    src = edge_index[0].astype(I32)
    dst = edge_index[1].astype(I32)
    eh = src.shape[0] // 2

    src_adj = src + (HALF - NU) * (src >= NU).astype(I32)
    pad_n = NT * EPT - eh

    def prep(sa, dl):
        sa = jnp.concatenate([sa, jnp.zeros((pad_n,), I32)])
        dl = jnp.concatenate([dl, jnp.full((pad_n,), NU, I32)])
        return (sa.reshape(NT, NBLK, BCH, CHK),
                dl.reshape(NT, NBLK, BCH, CHK))

    s0, d0 = prep(src_adj[eh:], dst[eh:])
    s1, d1 = prep(src_adj[:eh], dst[:eh] - NU)
    srcg = jnp.stack([s0, s1])
    dstg = jnp.stack([d0, d1])

    zpad = jnp.zeros((HALF - NU, D), F32)
    xp = jnp.concatenate([user_emb.astype(F32), zpad,
                          item_emb.astype(F32), zpad], axis=0)
    ur = users.astype(I32).reshape(NC, NT, 4, CHK)
    ir = (items.astype(I32) + HALF).reshape(NC, NT, 4, CHK)

    dinvb, z0 = _make_init()(dstg, xp)
    layer = _make_layer()
    z1 = layer(srcg, dstg, z0, dinvb)
    z2 = layer(srcg, dstg, z1, dinvb)
    z3 = layer(srcg, dstg, z2, dinvb)
    dvf = dinvb.reshape(NC * HALF, 16)
    gamma = _make_final()(xp, z1, z2, z3, dvf, ur, ir)
    return gamma.reshape(-1)
